# baseline (speedup 1.0000x reference)
import functools

import jax
import jax.numpy as jnp
from jax import lax
from jax.experimental import pallas as pl
from jax.experimental.pallas import tpu as pltpu
from jax.experimental.pallas import tpu_sc as plsc

B = 16384
EMB = 32
GEN = 26
HID = 128
HID2 = 64

_NC = 2
_NS = 16
_NW = _NC * _NS
_BPW = B // _NW


@functools.lru_cache(maxsize=None)
def _build_sc_gather():
    @functools.partial(
        pl.kernel,
        mesh=plsc.VectorSubcoreMesh(core_axis_name="c", subcore_axis_name="s"),
        compiler_params=pltpu.CompilerParams(use_tc_tiling_on_sc=False),
        out_type=[
            jax.ShapeDtypeStruct((B, EMB), jnp.float32),
            jax.ShapeDtypeStruct((B, EMB), jnp.bfloat16),
        ],
        scratch_types=[
            pltpu.VMEM((_BPW,), jnp.int32),
            pltpu.VMEM((_BPW,), jnp.int32),
            pltpu.VMEM((_BPW, EMB), jnp.float32),
            pltpu.VMEM((_BPW, EMB), jnp.bfloat16),
            pltpu.SemaphoreType.DMA,
            pltpu.SemaphoreType.DMA,
        ],
    )
    def _sc_gather(movie_table, user_table, movie_id, user_id,
                   movie_out, user_out,
                   idx_m, idx_u, rows_m, rows_u, sem_m, sem_u):
        wid = lax.axis_index("s") * _NC + lax.axis_index("c")
        base = wid * _BPW
        pltpu.sync_copy(movie_id.at[pl.ds(base, _BPW)], idx_m)
        pltpu.sync_copy(user_id.at[pl.ds(base, _BPW)], idx_u)
        cm = pltpu.async_copy(movie_table.at[idx_m], rows_m, sem_m)
        cu = pltpu.async_copy(user_table.at[idx_u], rows_u, sem_u)
        cm.wait()
        pltpu.sync_copy(rows_m, movie_out.at[pl.ds(base, _BPW)])
        cu.wait()
        pltpu.sync_copy(rows_u, user_out.at[pl.ds(base, _BPW)])

    return _sc_gather


def _mlp_body(me, ue, ge, w1, b1, w2, b2, w3, b3, out):
    w1m = w1[0:EMB, :]
    w1u = w1[EMB:2 * EMB, :]
    w1g = w1[2 * EMB:, :]
    h = (
        jnp.dot(me[:], w1m, preferred_element_type=jnp.float32)
        + jnp.dot(ue[:].astype(jnp.float32), w1u,
                  preferred_element_type=jnp.float32)
        + jnp.dot(ge[:], w1g, preferred_element_type=jnp.float32)
        + b1[:]
    )
    h = jnp.maximum(h, 0.0)
    h2 = jnp.maximum(jnp.dot(h, w2[:], preferred_element_type=jnp.float32) + b2[:], 0.0)
    out[:] = jnp.dot(h2, w3[:], preferred_element_type=jnp.float32) + b3[:]


def _mlp(me, ue, ge, w1, b1, w2, b2, w3, b3):
    bs = 2048
    grid = (B // bs,)
    const = lambda i: (0, 0)
    row = lambda i: (i, 0)
    return pl.pallas_call(
        _mlp_body,
        grid=grid,
        in_specs=[
            pl.BlockSpec((bs, EMB), row),
            pl.BlockSpec((bs, EMB), row),
            pl.BlockSpec((bs, GEN), row),
            pl.BlockSpec((2 * EMB + GEN, HID), const),
            pl.BlockSpec((1, HID), const),
            pl.BlockSpec((HID, HID2), const),
            pl.BlockSpec((1, HID2), const),
            pl.BlockSpec((HID2, 1), const),
            pl.BlockSpec((1, 1), const),
        ],
        out_specs=pl.BlockSpec((bs, 1), row),
        out_shape=jax.ShapeDtypeStruct((B, 1), jnp.float32),
    )(me, ue, ge, w1, b1, w2, b2, w3, b3)


def kernel(movie_id, user_id, genre_id, movie_table, user_table, W1, b1, W2, b2, W3, b3):
    movie_emb, user_emb = _build_sc_gather()(
        movie_table, user_table.astype(jnp.bfloat16), movie_id, user_id
    )
    return _mlp(
        movie_emb, user_emb, genre_id.astype(jnp.float32), W1,
        b1.reshape(1, HID), W2, b2.reshape(1, HID2), W3, b3.reshape(1, 1),
    )


# speedup vs baseline: 1.1615x; 1.1615x over previous
import functools

import jax
import jax.numpy as jnp
from jax import lax
from jax.experimental import pallas as pl
from jax.experimental.pallas import tpu as pltpu
from jax.experimental.pallas import tpu_sc as plsc

B = 16384
EMB = 32
GEN = 26
HID = 128
HID2 = 64

_NC = 2
_NS = 16
_NW = _NC * _NS
_BPW = B // _NW


@functools.lru_cache(maxsize=None)
def _build_sc_gather():
    @functools.partial(
        pl.kernel,
        mesh=plsc.VectorSubcoreMesh(core_axis_name="c", subcore_axis_name="s"),
        compiler_params=pltpu.CompilerParams(use_tc_tiling_on_sc=False),
        out_type=[
            jax.ShapeDtypeStruct((B, EMB), jnp.float32),
            jax.ShapeDtypeStruct((B, EMB), jnp.float32),
        ],
        scratch_types=[
            pltpu.VMEM((_BPW,), jnp.int32),
            pltpu.VMEM((_BPW,), jnp.int32),
            pltpu.VMEM((_BPW, EMB), jnp.float32),
            pltpu.VMEM((_BPW, EMB), jnp.float32),
            pltpu.SemaphoreType.DMA,
            pltpu.SemaphoreType.DMA,
        ],
    )
    def _sc_gather(movie_table, user_table, movie_id, user_id,
                   movie_out, user_out,
                   idx_m, idx_u, rows_m, rows_u, sem_m, sem_u):
        wid = lax.axis_index("s") * _NC + lax.axis_index("c")
        base = wid * _BPW
        pltpu.sync_copy(movie_id.at[pl.ds(base, _BPW)], idx_m)
        pltpu.sync_copy(user_id.at[pl.ds(base, _BPW)], idx_u)
        cm = pltpu.async_copy(movie_table.at[idx_m], rows_m, sem_m)
        cu = pltpu.async_copy(user_table.at[idx_u], rows_u, sem_u)
        cm.wait()
        pltpu.sync_copy(rows_m, movie_out.at[pl.ds(base, _BPW)])
        cu.wait()
        pltpu.sync_copy(rows_u, user_out.at[pl.ds(base, _BPW)])

    return _sc_gather


def _mlp_body(me, ue, ge, w1, b1, w2, b2, w3, b3, out):
    w1m = w1[0:EMB, :]
    w1u = w1[EMB:2 * EMB, :]
    w1g = w1[2 * EMB:, :]
    h = (
        jnp.dot(me[:], w1m, preferred_element_type=jnp.float32)
        + jnp.dot(ue[:], w1u, preferred_element_type=jnp.float32)
        + jnp.dot(ge[:], w1g, preferred_element_type=jnp.float32)
        + b1[:]
    )
    h = jnp.maximum(h, 0.0)
    h2 = jnp.maximum(jnp.dot(h, w2[:], preferred_element_type=jnp.float32) + b2[:], 0.0)
    out[:] = jnp.dot(h2, w3[:], preferred_element_type=jnp.float32) + b3[:]


def _mlp(me, ue, ge, w1, b1, w2, b2, w3, b3):
    bs = 2048
    grid = (B // bs,)
    const = lambda i: (0, 0)
    row = lambda i: (i, 0)
    return pl.pallas_call(
        _mlp_body,
        grid=grid,
        in_specs=[
            pl.BlockSpec((bs, EMB), row),
            pl.BlockSpec((bs, EMB), row),
            pl.BlockSpec((bs, GEN), row),
            pl.BlockSpec((2 * EMB + GEN, HID), const),
            pl.BlockSpec((1, HID), const),
            pl.BlockSpec((HID, HID2), const),
            pl.BlockSpec((1, HID2), const),
            pl.BlockSpec((HID2, 1), const),
            pl.BlockSpec((1, 1), const),
        ],
        out_specs=pl.BlockSpec((bs, 1), row),
        out_shape=jax.ShapeDtypeStruct((B, 1), jnp.float32),
    )(me, ue, ge, w1, b1, w2, b2, w3, b3)


def kernel(movie_id, user_id, genre_id, movie_table, user_table, W1, b1, W2, b2, W3, b3):
    movie_emb, user_emb = _build_sc_gather()(
        movie_table, user_table, movie_id, user_id
    )
    return _mlp(
        movie_emb, user_emb, genre_id.astype(jnp.float32), W1,
        b1.reshape(1, HID), W2, b2.reshape(1, HID2), W3, b3.reshape(1, 1),
    )
